# Initial kernel scaffold; baseline (speedup 1.0000x reference)
import jax
import jax.numpy as jnp
from jax.experimental import pallas as pl


def kernel(x, edge_index, W, b, gamma, beta, W_out, b_out):
    raise NotImplementedError("write your pallas kernel here")


# speedup vs baseline: 1.8956x; 1.8956x over previous
import jax
import jax.numpy as jnp
from jax.experimental import pallas as pl

N = 10000
NPAD = 10240
D = 256
L = 5
OUT = 3
EPS = 1e-5
BR = 1024


def _mm_body(h_ref, w_ref, o_ref):
    o_ref[...] = jnp.dot(h_ref[...], w_ref[...],
                         preferred_element_type=jnp.float32)


def _mm(h, wt):
    return pl.pallas_call(
        _mm_body,
        grid=(NPAD // BR,),
        in_specs=[
            pl.BlockSpec((BR, D), lambda i: (i, 0)),
            pl.BlockSpec((D, D), lambda i: (0, 0)),
        ],
        out_specs=pl.BlockSpec((BR, D), lambda i: (i, 0)),
        out_shape=jax.ShapeDtypeStruct((NPAD, D), jnp.float32),
    )(h, wt)


def kernel(x, edge_index, W, b, gamma, beta, W_out, b_out):
    row = edge_index[0]
    col = edge_index[1]
    deg = jax.ops.segment_sum(jnp.ones_like(col, jnp.float32), col,
                              num_segments=N) + 1.0
    dis = jax.lax.rsqrt(deg)

    xp = jnp.pad(x, ((0, NPAD - N), (0, 0)))
    disp = jnp.pad(dis, (0, NPAD - N), constant_values=1.0)
    scale = 1.0 / jnp.sqrt(1.0 + EPS)

    h = xp
    for i in range(L):
        residual = h
        z = _mm(h, W[i].T) * disp[:, None]
        acc = jax.ops.segment_sum(z[row], col, num_segments=N)
        acc = jnp.pad(acc, ((0, NPAD - N), (0, 0)))
        conv = disp[:, None] * (acc + z) + b[i]
        a = jnp.where(conv > 0, conv, 0.2 * conv)
        a = a * (gamma[i] * scale) + beta[i]
        h = a + residual
    zf = h * disp[:, None]
    accf = jax.ops.segment_sum(zf[row], col, num_segments=N)
    accf = jnp.pad(accf, ((0, NPAD - N), (0, 0)))
    aggf = disp[:, None] * (accf + zf)
    wof = jnp.zeros((D, 128), jnp.float32).at[:, :OUT].set(W_out.T)
    dxp = _mm_out(aggf, wof) + jnp.pad(b_out, (0, 128 - OUT))
    return dxp[:N, :OUT]


def _mm_out(h, wt):
    return pl.pallas_call(
        _mm_body,
        grid=(NPAD // BR,),
        in_specs=[
            pl.BlockSpec((BR, D), lambda i: (i, 0)),
            pl.BlockSpec((D, 128), lambda i: (0, 0)),
        ],
        out_specs=pl.BlockSpec((BR, 128), lambda i: (i, 0)),
        out_shape=jax.ShapeDtypeStruct((NPAD, 128), jnp.float32),
    )(h, wt)


# speedup vs baseline: 4.1797x; 2.2050x over previous
import functools

import jax
import jax.numpy as jnp
from jax import lax
from jax.experimental import pallas as pl
from jax.experimental.pallas import tpu as pltpu
from jax.experimental.pallas import tpu_sc as plsc

N = 10000
NPAD = 10240
D = 256
HW = D // 2
L = 5
OUT = 3
EPS = 1e-5
BR = 1024

NC = 2
NS = 16
CH = 128
E = 160000
EPT = 10112
NCHUNK = EPT // CH
EPAD = EPT * NS
HALF = N // NC
TRASH = 64
SROWS = 5120
ZR = 16

_mesh = plsc.VectorSubcoreMesh(core_axis_name="c", subcore_axis_name="s",
                               num_cores=NC, num_subcores=NS)


def _local_indices(colbuf, lidx, half_base):
    for i in range(CH // 16):
        c = colbuf[pl.ds(i * 16, 16)]
        l = c - half_base
        ok = (l >= 0) & (l < HALF)
        lidx[pl.ds(i * 16, 16)] = jnp.where(ok, l, HALF + (c & (TRASH - 1)))


@functools.partial(
    pl.kernel,
    out_type=jax.ShapeDtypeStruct((NPAD, HW), jnp.float32),
    mesh=_mesh,
    scratch_types=[
        pltpu.VMEM((CH,), jnp.int32),
        pltpu.VMEM((CH,), jnp.int32),
        pltpu.VMEM((CH, HW), jnp.float32),
        pltpu.VMEM((ZR, HW), jnp.float32),
        pltpu.VMEM_SHARED((SROWS, HW), jnp.float32),
    ],
)
def _sc_degree(colp_hbm, ones_hbm, zeros_hbm, deg_hbm,
               colbuf, lidx, ones, zbuf, spdeg):
    s = lax.axis_index("c")
    t = lax.axis_index("s")
    pltpu.sync_copy(ones_hbm, ones)
    pltpu.sync_copy(zeros_hbm, zbuf)
    for j in range(SROWS // NS // ZR):
        pltpu.sync_copy(zbuf, spdeg.at[pl.ds(t * (SROWS // NS) + j * ZR, ZR)])
    plsc.subcore_barrier()
    half_base = s * HALF

    def chunk(k, _):
        base = t * EPT + k * CH
        pltpu.sync_copy(colp_hbm.at[pl.ds(base, CH)], colbuf)
        _local_indices(colbuf, lidx, half_base)
        pltpu.sync_copy(ones, spdeg.at[lidx], add=True)
        return 0

    lax.fori_loop(0, NCHUNK, chunk, 0)
    plsc.subcore_barrier()
    base = s * HALF
    pltpu.sync_copy(spdeg.at[pl.ds(t * 312, 312)],
                    deg_hbm.at[pl.ds(base + t * 312, 312)])

    @pl.when(t == NS - 1)
    def _():
        pltpu.sync_copy(spdeg.at[pl.ds(4992, 8)],
                        deg_hbm.at[pl.ds(base + 4992, 8)])

    @pl.when((t == NS - 1) & (s == NC - 1))
    def _():
        for j in range((NPAD - N) // ZR):
            pltpu.sync_copy(zbuf, deg_hbm.at[pl.ds(N + j * ZR, ZR)])


@functools.partial(
    pl.kernel,
    out_type=(jax.ShapeDtypeStruct((NPAD, HW), jnp.float32),
              jax.ShapeDtypeStruct((NPAD, HW), jnp.float32)),
    mesh=_mesh,
    scratch_types=[
        pltpu.VMEM((CH,), jnp.int32),
        pltpu.VMEM((CH,), jnp.int32),
        pltpu.VMEM((CH,), jnp.int32),
        pltpu.VMEM((CH, HW), jnp.float32),
        pltpu.VMEM((CH, HW), jnp.float32),
        pltpu.VMEM((ZR, HW), jnp.float32),
        pltpu.VMEM_SHARED((SROWS, HW), jnp.float32),
        pltpu.VMEM_SHARED((SROWS, HW), jnp.float32),
        pltpu.SemaphoreType.DMA,
        pltpu.SemaphoreType.DMA,
    ],
)
def _sc_segsum(zl_hbm, zr_hbm, rowp_hbm, colp_hbm, zeros_hbm,
               accl_hbm, accr_hbm,
               rowbuf, colbuf, lidx, gl, gr, zbuf, accl, accr, seml, semr):
    s = lax.axis_index("c")
    t = lax.axis_index("s")
    pltpu.sync_copy(zeros_hbm, zbuf)
    for j in range(SROWS // NS // ZR):
        off = t * (SROWS // NS) + j * ZR
        pltpu.sync_copy(zbuf, accl.at[pl.ds(off, ZR)])
        pltpu.sync_copy(zbuf, accr.at[pl.ds(off, ZR)])
    plsc.subcore_barrier()
    half_base = s * HALF

    def chunk(k, _):
        base = t * EPT + k * CH
        pltpu.sync_copy(colp_hbm.at[pl.ds(base, CH)], colbuf)
        pltpu.sync_copy(rowp_hbm.at[pl.ds(base, CH)], rowbuf)
        _local_indices(colbuf, lidx, half_base)
        cl = pltpu.async_copy(zl_hbm.at[rowbuf], gl, seml)
        cr = pltpu.async_copy(zr_hbm.at[rowbuf], gr, semr)
        cl.wait()
        pltpu.sync_copy(gl, accl.at[lidx], add=True)
        cr.wait()
        pltpu.sync_copy(gr, accr.at[lidx], add=True)
        return 0

    lax.fori_loop(0, NCHUNK, chunk, 0)
    plsc.subcore_barrier()
    base = s * HALF
    pltpu.sync_copy(accl.at[pl.ds(t * 312, 312)],
                    accl_hbm.at[pl.ds(base + t * 312, 312)])
    pltpu.sync_copy(accr.at[pl.ds(t * 312, 312)],
                    accr_hbm.at[pl.ds(base + t * 312, 312)])

    @pl.when(t == NS - 1)
    def _():
        pltpu.sync_copy(accl.at[pl.ds(4992, 8)],
                        accl_hbm.at[pl.ds(base + 4992, 8)])
        pltpu.sync_copy(accr.at[pl.ds(4992, 8)],
                        accr_hbm.at[pl.ds(base + 4992, 8)])

    @pl.when((t == NS - 1) & (s == NC - 1))
    def _():
        for j in range((NPAD - N) // ZR):
            pltpu.sync_copy(zbuf, accl_hbm.at[pl.ds(N + j * ZR, ZR)])
            pltpu.sync_copy(zbuf, accr_hbm.at[pl.ds(N + j * ZR, ZR)])


def _dis(deg_ref):
    return lax.rsqrt(deg_ref[:, 0:1] + 1.0)


def _split_z(z, zl_ref, zr_ref):
    zl_ref[...] = z[:, 0:HW]
    zr_ref[...] = z[:, HW:D]


def _tc_first_body(h_ref, w_ref, deg_ref, zl_ref, zr_ref):
    z = _dis(deg_ref) * jnp.dot(h_ref[...], w_ref[...],
                                preferred_element_type=jnp.float32)
    _split_z(z, zl_ref, zr_ref)


def _epilogue(accl_ref, accr_ref, zl_ref, zr_ref, deg_ref,
              b_ref, gs_ref, beta_ref):
    dis = _dis(deg_ref)
    conv = dis * jnp.concatenate(
        [accl_ref[...] + zl_ref[...], accr_ref[...] + zr_ref[...]], axis=1)
    conv = conv + b_ref[0:1, :]
    a = jnp.where(conv > 0, conv, 0.2 * conv)
    return a * gs_ref[0:1, :] + beta_ref[0:1, :], dis


def _tc_mid_body(accl_ref, accr_ref, zl_ref, zr_ref, hp_ref, w_ref, deg_ref,
                 b_ref, gs_ref, beta_ref, h_ref, zlo_ref, zro_ref):
    a, dis = _epilogue(accl_ref, accr_ref, zl_ref, zr_ref, deg_ref,
                       b_ref, gs_ref, beta_ref)
    h = a + hp_ref[...]
    h_ref[...] = h
    z = dis * jnp.dot(h, w_ref[...], preferred_element_type=jnp.float32)
    _split_z(z, zlo_ref, zro_ref)


def _tc_last_body(accl_ref, accr_ref, zl_ref, zr_ref, hp_ref, deg_ref,
                  b_ref, gs_ref, beta_ref, zlo_ref, zro_ref):
    a, dis = _epilogue(accl_ref, accr_ref, zl_ref, zr_ref, deg_ref,
                       b_ref, gs_ref, beta_ref)
    z = dis * (a + hp_ref[...])
    _split_z(z, zlo_ref, zro_ref)


def _tc_out_body(accl_ref, accr_ref, zl_ref, zr_ref, deg_ref,
                 w_ref, b_ref, dx_ref):
    agg = _dis(deg_ref) * jnp.concatenate(
        [accl_ref[...] + zl_ref[...], accr_ref[...] + zr_ref[...]], axis=1)
    dx_ref[...] = jnp.dot(agg, w_ref[...],
                          preferred_element_type=jnp.float32) + b_ref[0:1, :]


def _row_spec(width=D):
    return pl.BlockSpec((BR, width), lambda i: (i, 0))


def _full_spec(shape):
    return pl.BlockSpec(shape, lambda i: (0, 0))


def _pcall(body, in_specs, out_widths):
    outs = tuple(jax.ShapeDtypeStruct((NPAD, w), jnp.float32)
                 for w in out_widths)
    out_specs = tuple(_row_spec(w) for w in out_widths)
    if len(out_widths) == 1:
        outs, out_specs = outs[0], out_specs[0]
    return pl.pallas_call(body, grid=(NPAD // BR,), in_specs=in_specs,
                          out_specs=out_specs, out_shape=outs)


def kernel(x, edge_index, W, b, gamma, beta, W_out, b_out):
    row = edge_index[0]
    col = edge_index[1]
    rowp = jnp.pad(row, (0, EPAD - E))
    colp = jnp.pad(col, (0, EPAD - E), constant_values=N)
    xp = jnp.pad(x, ((0, NPAD - N), (0, 0)))
    Wt = jnp.transpose(W, (0, 2, 1))
    scale = 1.0 / jnp.sqrt(1.0 + EPS)
    gs = gamma * scale
    b8 = jnp.broadcast_to(b[:, None, :], (L, 8, D))
    gs8 = jnp.broadcast_to(gs[:, None, :], (L, 8, D))
    beta8 = jnp.broadcast_to(beta[:, None, :], (L, 8, D))
    wof = jnp.zeros((D, 128), jnp.float32).at[:, :OUT].set(W_out.T)
    bof = jnp.broadcast_to(jnp.pad(b_out, (0, 128 - OUT))[None, :], (8, 128))

    ones128 = jnp.ones((CH, HW), jnp.float32)
    zeroshw = jnp.zeros((ZR, HW), jnp.float32)

    deg16 = _sc_degree(colp, ones128, zeroshw)

    par_spec = _full_spec((8, D))
    deg_spec = _row_spec(HW)
    half_spec = _row_spec(HW)

    zl, zr = _pcall(_tc_first_body,
                    [_row_spec(), _full_spec((D, D)), deg_spec],
                    (HW, HW))(xp, Wt[0], deg16)
    h = xp
    for i in range(1, L):
        accl, accr = _sc_segsum(zl, zr, rowp, colp, zeroshw)
        h, zl, zr = _pcall(_tc_mid_body,
                           [half_spec, half_spec, half_spec, half_spec,
                            _row_spec(), _full_spec((D, D)), deg_spec,
                            par_spec, par_spec, par_spec], (D, HW, HW))(
            accl, accr, zl, zr, h, Wt[i], deg16,
            b8[i - 1], gs8[i - 1], beta8[i - 1])
    accl, accr = _sc_segsum(zl, zr, rowp, colp, zeroshw)
    zl, zr = _pcall(_tc_last_body,
                    [half_spec, half_spec, half_spec, half_spec,
                     _row_spec(), deg_spec,
                     par_spec, par_spec, par_spec], (HW, HW))(
        accl, accr, zl, zr, h, deg16, b8[L - 1], gs8[L - 1], beta8[L - 1])
    accl, accr = _sc_segsum(zl, zr, rowp, colp, zeroshw)
    dxp = _pcall(_tc_out_body,
                 [half_spec, half_spec, half_spec, half_spec, deg_spec,
                  _full_spec((D, 128)), _full_spec((8, 128))], (128,))(
        accl, accr, zl, zr, deg16, wof, bof)
    return dxp[:N, :OUT]


# speedup vs baseline: 10.0527x; 2.4051x over previous
import functools

import jax
import jax.numpy as jnp
from jax import lax
from jax.experimental import pallas as pl
from jax.experimental.pallas import tpu as pltpu
from jax.experimental.pallas import tpu_sc as plsc

N = 10000
NPAD = 10240
D = 256
HW = D // 2
L = 5
OUT = 3
EPS = 1e-5
BR = 1024

NC = 2
NS = 16
CH = 128
E = 160000
EPT = 10112
NCHUNK = EPT // CH
EPAD = EPT * NS
HALF = N // NC
TRASH = 64
SROWS = 5120
ZR = 16

_mesh = plsc.VectorSubcoreMesh(core_axis_name="c", subcore_axis_name="s",
                               num_cores=NC, num_subcores=NS)


def _local_indices(colbuf, lidx, half_base):
    for i in range(CH // 16):
        c = colbuf[pl.ds(i * 16, 16)]
        l = c - half_base
        ok = (l >= 0) & (l < HALF)
        lidx[pl.ds(i * 16, 16)] = jnp.where(ok, l, HALF + (c & (TRASH - 1)))


@functools.partial(
    pl.kernel,
    out_type=jax.ShapeDtypeStruct((NPAD, HW), jnp.float32),
    mesh=_mesh,
    scratch_types=[
        pltpu.VMEM((CH,), jnp.int32),
        pltpu.VMEM((CH,), jnp.int32),
        pltpu.VMEM((CH, HW), jnp.float32),
        pltpu.VMEM((ZR, HW), jnp.float32),
        pltpu.VMEM_SHARED((SROWS, HW), jnp.float32),
    ],
)
def _sc_degree(colp_hbm, ones_hbm, zeros_hbm, deg_hbm,
               colbuf, lidx, ones, zbuf, spdeg):
    s = lax.axis_index("c")
    t = lax.axis_index("s")
    pltpu.sync_copy(ones_hbm, ones)
    pltpu.sync_copy(zeros_hbm, zbuf)
    for j in range(SROWS // NS // ZR):
        pltpu.sync_copy(zbuf, spdeg.at[pl.ds(t * (SROWS // NS) + j * ZR, ZR)])
    plsc.subcore_barrier()
    half_base = s * HALF

    def chunk(k, _):
        base = t * EPT + k * CH
        pltpu.sync_copy(colp_hbm.at[pl.ds(base, CH)], colbuf)
        _local_indices(colbuf, lidx, half_base)
        pltpu.sync_copy(ones, spdeg.at[lidx], add=True)
        return 0

    lax.fori_loop(0, NCHUNK, chunk, 0)
    plsc.subcore_barrier()
    base = s * HALF
    pltpu.sync_copy(spdeg.at[pl.ds(t * 312, 312)],
                    deg_hbm.at[pl.ds(base + t * 312, 312)])

    @pl.when(t == NS - 1)
    def _():
        pltpu.sync_copy(spdeg.at[pl.ds(4992, 8)],
                        deg_hbm.at[pl.ds(base + 4992, 8)])

    @pl.when((t == NS - 1) & (s == NC - 1))
    def _():
        for j in range((NPAD - N) // ZR):
            pltpu.sync_copy(zbuf, deg_hbm.at[pl.ds(N + j * ZR, ZR)])


EPAD2 = EPAD + CH


@functools.partial(
    pl.kernel,
    out_type=jax.ShapeDtypeStruct((2 * NPAD, HW), jnp.float32),
    mesh=_mesh,
    scratch_types=[
        pltpu.VMEM((CH,), jnp.int32),
        pltpu.VMEM((CH,), jnp.int32),
        pltpu.VMEM((CH,), jnp.int32),
        pltpu.VMEM((CH,), jnp.int32),
        pltpu.VMEM((CH, HW), jnp.float32),
        pltpu.VMEM((CH, HW), jnp.float32),
        pltpu.VMEM((ZR, HW), jnp.float32),
        pltpu.VMEM_SHARED((NPAD, HW), jnp.float32),
        pltpu.SemaphoreType.DMA,
        pltpu.SemaphoreType.DMA,
    ],
)
def _sc_segsum(zs_hbm, rowp_hbm, colp_hbm, zeros_hbm, accs_hbm,
               iAr, iAc, iBr, iBc, gA, gB, zbuf, spacc, semA, semB):
    s = lax.axis_index("c")
    t = lax.axis_index("s")
    pltpu.sync_copy(zeros_hbm, zbuf)
    rpt = NPAD // NS
    for j in range(rpt // ZR):
        pltpu.sync_copy(zbuf, spacc.at[pl.ds(t * rpt + j * ZR, ZR)])
    plsc.subcore_barrier()

    def load_idx(k, ir, ic):
        base = t * EPT + k * CH
        pltpu.sync_copy(rowp_hbm.at[pl.ds(base, CH)], ir)
        pltpu.sync_copy(colp_hbm.at[pl.ds(base, CH)], ic)
        for i in range(CH // 16):
            r = ir[pl.ds(i * 16, 16)]
            ir[pl.ds(i * 16, 16)] = r * 2 + s

    def start_gather(ir, g, sem):
        pltpu.async_copy(zs_hbm.at[ir], g, sem)

    def wait_gather(g, sem):
        pltpu.make_async_copy(zs_hbm.at[pl.ds(0, CH)], g, sem).wait()

    load_idx(0, iAr, iAc)
    start_gather(iAr, gA, semA)
    load_idx(1, iBr, iBc)

    def pair(k2, _):
        a = 2 * k2
        start_gather(iBr, gB, semB)
        wait_gather(gA, semA)
        pltpu.sync_copy(gA, spacc.at[iAc], add=True)
        load_idx(a + 2, iAr, iAc)
        start_gather(iAr, gA, semA)
        wait_gather(gB, semB)
        pltpu.sync_copy(gB, spacc.at[iBc], add=True)
        load_idx(a + 3, iBr, iBc)
        return 0

    lax.fori_loop(0, (NCHUNK - 1) // 2, pair, 0)
    wait_gather(gA, semA)
    pltpu.sync_copy(gA, spacc.at[iAc], add=True)
    plsc.subcore_barrier()
    pltpu.sync_copy(spacc.at[pl.ds(t * rpt, rpt)],
                    accs_hbm.at[pl.ds(s * NPAD + t * rpt, rpt)])


def _dis(deg_ref):
    return lax.rsqrt(deg_ref[:, 0:1] + 1.0)


def _tc_first_body(h_ref, w_ref, deg_ref, z_ref):
    z_ref[...] = _dis(deg_ref) * jnp.dot(h_ref[...], w_ref[...],
                                         preferred_element_type=jnp.float32)


def _epilogue(accl_ref, accr_ref, zp_ref, deg_ref, b_ref, gs_ref, beta_ref):
    dis = _dis(deg_ref)
    zp = zp_ref[...]
    conv = dis * jnp.concatenate(
        [accl_ref[...] + zp[:, 0:HW], accr_ref[...] + zp[:, HW:D]], axis=1)
    conv = conv + b_ref[0:1, :]
    a = jnp.where(conv > 0, conv, 0.2 * conv)
    return a * gs_ref[0:1, :] + beta_ref[0:1, :], dis


def _tc_mid_body(accl_ref, accr_ref, zp_ref, hp_ref, w_ref, deg_ref,
                 b_ref, gs_ref, beta_ref, h_ref, z_ref):
    a, dis = _epilogue(accl_ref, accr_ref, zp_ref, deg_ref,
                       b_ref, gs_ref, beta_ref)
    h = a + hp_ref[...]
    h_ref[...] = h
    z_ref[...] = dis * jnp.dot(h, w_ref[...],
                               preferred_element_type=jnp.float32)


def _tc_last_body(accl_ref, accr_ref, zp_ref, hp_ref, deg_ref,
                  b_ref, gs_ref, beta_ref, z_ref):
    a, dis = _epilogue(accl_ref, accr_ref, zp_ref, deg_ref,
                       b_ref, gs_ref, beta_ref)
    z_ref[...] = dis * (a + hp_ref[...])


def _tc_out_body(accl_ref, accr_ref, zp_ref, deg_ref, w_ref, b_ref, dx_ref):
    dis = _dis(deg_ref)
    zp = zp_ref[...]
    agg = dis * jnp.concatenate(
        [accl_ref[...] + zp[:, 0:HW], accr_ref[...] + zp[:, HW:D]], axis=1)
    dx_ref[...] = jnp.dot(agg, w_ref[...],
                          preferred_element_type=jnp.float32) + b_ref[0:1, :]


def _row_spec(width=D):
    return pl.BlockSpec((BR, width), lambda i: (i, 0))


_accl_spec = pl.BlockSpec((BR, HW), lambda i: (i, 0))
_accr_spec = pl.BlockSpec((BR, HW), lambda i: (i + NPAD // BR, 0))


def _full_spec(shape):
    return pl.BlockSpec(shape, lambda i: (0, 0))


def _pcall(body, in_specs, out_widths):
    outs = tuple(jax.ShapeDtypeStruct((NPAD, w), jnp.float32)
                 for w in out_widths)
    out_specs = tuple(_row_spec(w) for w in out_widths)
    if len(out_widths) == 1:
        outs, out_specs = outs[0], out_specs[0]
    return pl.pallas_call(body, grid=(NPAD // BR,), in_specs=in_specs,
                          out_specs=out_specs, out_shape=outs)


def kernel(x, edge_index, W, b, gamma, beta, W_out, b_out):
    row = edge_index[0]
    col = edge_index[1]
    pr = (jnp.arange(EPAD2 - E, dtype=jnp.int32) * 37) % N
    pc = N + (jnp.arange(EPAD2 - E, dtype=jnp.int32) % (NPAD - N))
    rowp = jnp.concatenate([row, pr])
    colp = jnp.concatenate([col, pc])
    xp = jnp.pad(x, ((0, NPAD - N), (0, 0)))
    Wt = jnp.transpose(W, (0, 2, 1))
    scale = 1.0 / jnp.sqrt(1.0 + EPS)
    gs = gamma * scale
    b8 = jnp.broadcast_to(b[:, None, :], (L, 8, D))
    gs8 = jnp.broadcast_to(gs[:, None, :], (L, 8, D))
    beta8 = jnp.broadcast_to(beta[:, None, :], (L, 8, D))
    wof = jnp.zeros((D, 128), jnp.float32).at[:, :OUT].set(W_out.T)
    bof = jnp.broadcast_to(jnp.pad(b_out, (0, 128 - OUT))[None, :], (8, 128))

    ones128 = jnp.ones((CH, HW), jnp.float32)
    zeroshw = jnp.zeros((ZR, HW), jnp.float32)

    deg16 = _sc_degree(colp, ones128, zeroshw)

    par_spec = _full_spec((8, D))
    deg_spec = _row_spec(HW)

    def seg(z):
        return _sc_segsum(z.reshape(2 * NPAD, HW), rowp, colp, zeroshw)

    z = _pcall(_tc_first_body,
               [_row_spec(), _full_spec((D, D)), deg_spec],
               (D,))(xp, Wt[0], deg16)
    h = xp
    for i in range(1, L):
        accs = seg(z)
        h, z = _pcall(_tc_mid_body,
                      [_accl_spec, _accr_spec, _row_spec(), _row_spec(),
                       _full_spec((D, D)), deg_spec,
                       par_spec, par_spec, par_spec], (D, D))(
            accs, accs, z, h, Wt[i], deg16,
            b8[i - 1], gs8[i - 1], beta8[i - 1])
    accs = seg(z)
    z = _pcall(_tc_last_body,
               [_accl_spec, _accr_spec, _row_spec(), _row_spec(), deg_spec,
                par_spec, par_spec, par_spec], (D,))(
        accs, accs, z, h, deg16, b8[L - 1], gs8[L - 1], beta8[L - 1])
    accs = seg(z)
    dxp = _pcall(_tc_out_body,
                 [_accl_spec, _accr_spec, _row_spec(), deg_spec,
                  _full_spec((D, 128)), _full_spec((8, 128))], (128,))(
        accs, accs, z, deg16, wof, bof)
    return dxp[:N, :OUT]


# speedup vs baseline: 11.6777x; 1.1617x over previous
import functools

import jax
import jax.numpy as jnp
from jax import lax
from jax.experimental import pallas as pl
from jax.experimental.pallas import tpu as pltpu
from jax.experimental.pallas import tpu_sc as plsc

N = 10000
NPAD = 10240
D = 256
HW = D // 2
L = 5
OUT = 3
EPS = 1e-5
BR = 1024

NC = 2
NS = 16
CH = 128
E = 160000
EPT = 10240
NCHUNK = EPT // CH
EPAD = EPT * NS
HALF = N // NC
TRASH = 64
SROWS = 5120
ZR = 16

_mesh = plsc.VectorSubcoreMesh(core_axis_name="c", subcore_axis_name="s",
                               num_cores=NC, num_subcores=NS)


def _local_indices(colbuf, lidx, half_base):
    for i in range(CH // 16):
        c = colbuf[pl.ds(i * 16, 16)]
        l = c - half_base
        ok = (l >= 0) & (l < HALF)
        lidx[pl.ds(i * 16, 16)] = jnp.where(ok, l, HALF + (c & (TRASH - 1)))


@functools.partial(
    pl.kernel,
    out_type=jax.ShapeDtypeStruct((NPAD, HW), jnp.float32),
    mesh=_mesh,
    scratch_types=[
        pltpu.VMEM((CH,), jnp.int32),
        pltpu.VMEM((CH,), jnp.int32),
        pltpu.VMEM((CH, HW), jnp.float32),
        pltpu.VMEM((ZR, HW), jnp.float32),
        pltpu.VMEM_SHARED((SROWS, HW), jnp.float32),
    ],
)
def _sc_degree(colp_hbm, ones_hbm, zeros_hbm, deg_hbm,
               colbuf, lidx, ones, zbuf, spdeg):
    s = lax.axis_index("c")
    t = lax.axis_index("s")
    pltpu.sync_copy(ones_hbm, ones)
    pltpu.sync_copy(zeros_hbm, zbuf)
    for j in range(SROWS // NS // ZR):
        pltpu.sync_copy(zbuf, spdeg.at[pl.ds(t * (SROWS // NS) + j * ZR, ZR)])
    plsc.subcore_barrier()
    half_base = s * HALF

    def chunk(k, _):
        base = t * EPT + k * CH
        pltpu.sync_copy(colp_hbm.at[pl.ds(base, CH)], colbuf)
        _local_indices(colbuf, lidx, half_base)
        pltpu.sync_copy(ones, spdeg.at[lidx], add=True)
        return 0

    lax.fori_loop(0, NCHUNK, chunk, 0)
    plsc.subcore_barrier()
    base = s * HALF
    pltpu.sync_copy(spdeg.at[pl.ds(t * 312, 312)],
                    deg_hbm.at[pl.ds(base + t * 312, 312)])

    @pl.when(t == NS - 1)
    def _():
        pltpu.sync_copy(spdeg.at[pl.ds(4992, 8)],
                        deg_hbm.at[pl.ds(base + 4992, 8)])

    @pl.when((t == NS - 1) & (s == NC - 1))
    def _():
        for j in range((NPAD - N) // ZR):
            pltpu.sync_copy(zbuf, deg_hbm.at[pl.ds(N + j * ZR, ZR)])


@functools.partial(
    pl.kernel,
    out_type=jax.ShapeDtypeStruct((2 * NPAD, HW), jnp.float32),
    mesh=_mesh,
    scratch_types=[
        pltpu.VMEM((CH,), jnp.int32),
        pltpu.VMEM((CH,), jnp.int32),
        pltpu.VMEM((CH,), jnp.int32),
        pltpu.VMEM((CH,), jnp.int32),
        pltpu.VMEM((CH, HW), jnp.float32),
        pltpu.VMEM((CH, HW), jnp.float32),
        pltpu.VMEM((ZR, HW), jnp.float32),
        pltpu.VMEM_SHARED((NPAD, HW), jnp.float32),
        pltpu.SemaphoreType.DMA,
        pltpu.SemaphoreType.DMA,
        pltpu.SemaphoreType.DMA,
        pltpu.SemaphoreType.DMA,
    ],
)
def _sc_segsum(zs_hbm, rows2_hbm, cols2_hbm, zeros_hbm, accs_hbm,
               iAr, iAc, iBr, iBc, gA, gB, zbuf, spacc,
               sGA, sGB, sSA, sSB):
    s = lax.axis_index("c")
    t = lax.axis_index("s")
    pltpu.sync_copy(zeros_hbm, zbuf)
    rpt = NPAD // NS
    for j in range(rpt // ZR):
        pltpu.sync_copy(zbuf, spacc.at[pl.ds(t * rpt + j * ZR, ZR)])
    plsc.subcore_barrier()
    rbase = (s * NS + t) * EPT
    cbase = t * EPT

    def load_idx(k, ir, ic):
        pltpu.sync_copy(rows2_hbm.at[pl.ds(rbase + k * CH, CH)], ir)
        pltpu.sync_copy(cols2_hbm.at[pl.ds(cbase + k * CH, CH)], ic)

    def wait_dma(g, sem):
        pltpu.make_async_copy(zs_hbm.at[pl.ds(0, CH)], g, sem).wait()

    load_idx(0, iAr, iAc)
    pltpu.async_copy(zs_hbm.at[iAr], gA, sGA)
    load_idx(1, iBr, iBc)
    pltpu.async_copy(zs_hbm.at[iBr], gB, sGB)

    def pair(k2, _):
        a = 2 * k2
        wait_dma(gA, sGA)
        pltpu.async_copy(gA, spacc.at[iAc], sSA, add=True)
        load_idx(a + 2, iAr, iAc)
        wait_dma(gA, sSA)
        pltpu.async_copy(zs_hbm.at[iAr], gA, sGA)
        wait_dma(gB, sGB)
        pltpu.async_copy(gB, spacc.at[iBc], sSB, add=True)
        load_idx(a + 3, iBr, iBc)
        wait_dma(gB, sSB)
        pltpu.async_copy(zs_hbm.at[iBr], gB, sGB)
        return 0

    lax.fori_loop(0, NCHUNK // 2 - 1, pair, 0)
    wait_dma(gA, sGA)
    pltpu.sync_copy(gA, spacc.at[iAc], add=True)
    wait_dma(gB, sGB)
    pltpu.sync_copy(gB, spacc.at[iBc], add=True)
    plsc.subcore_barrier()
    pltpu.sync_copy(spacc.at[pl.ds(t * rpt, rpt)],
                    accs_hbm.at[pl.ds(s * NPAD + t * rpt, rpt)])


def _dis(deg_ref):
    return lax.rsqrt(deg_ref[:, 0:1] + 1.0)


def _tc_first_body(h_ref, w_ref, deg_ref, z_ref):
    z_ref[...] = _dis(deg_ref) * jnp.dot(h_ref[...], w_ref[...],
                                         preferred_element_type=jnp.float32)


def _epilogue(accl_ref, accr_ref, zp_ref, deg_ref, b_ref, gs_ref, beta_ref):
    dis = _dis(deg_ref)
    zp = zp_ref[...]
    conv = dis * jnp.concatenate(
        [accl_ref[...] + zp[:, 0:HW], accr_ref[...] + zp[:, HW:D]], axis=1)
    conv = conv + b_ref[0:1, :]
    a = jnp.where(conv > 0, conv, 0.2 * conv)
    return a * gs_ref[0:1, :] + beta_ref[0:1, :], dis


def _tc_mid_body(accl_ref, accr_ref, zp_ref, hp_ref, w_ref, deg_ref,
                 b_ref, gs_ref, beta_ref, h_ref, z_ref):
    a, dis = _epilogue(accl_ref, accr_ref, zp_ref, deg_ref,
                       b_ref, gs_ref, beta_ref)
    h = a + hp_ref[...]
    h_ref[...] = h
    z_ref[...] = dis * jnp.dot(h, w_ref[...],
                               preferred_element_type=jnp.float32)


def _tc_last_body(accl_ref, accr_ref, zp_ref, hp_ref, deg_ref,
                  b_ref, gs_ref, beta_ref, z_ref):
    a, dis = _epilogue(accl_ref, accr_ref, zp_ref, deg_ref,
                       b_ref, gs_ref, beta_ref)
    z_ref[...] = dis * (a + hp_ref[...])


def _tc_out_body(accl_ref, accr_ref, zp_ref, deg_ref, w_ref, b_ref, dx_ref):
    dis = _dis(deg_ref)
    zp = zp_ref[...]
    agg = dis * jnp.concatenate(
        [accl_ref[...] + zp[:, 0:HW], accr_ref[...] + zp[:, HW:D]], axis=1)
    dx_ref[...] = jnp.dot(agg, w_ref[...],
                          preferred_element_type=jnp.float32) + b_ref[0:1, :]


def _row_spec(width=D):
    return pl.BlockSpec((BR, width), lambda i: (i, 0))


_accl_spec = pl.BlockSpec((BR, HW), lambda i: (i, 0))
_accr_spec = pl.BlockSpec((BR, HW), lambda i: (i + NPAD // BR, 0))


def _full_spec(shape):
    return pl.BlockSpec(shape, lambda i: (0, 0))


def _pcall(body, in_specs, out_widths):
    outs = tuple(jax.ShapeDtypeStruct((NPAD, w), jnp.float32)
                 for w in out_widths)
    out_specs = tuple(_row_spec(w) for w in out_widths)
    if len(out_widths) == 1:
        outs, out_specs = outs[0], out_specs[0]
    return pl.pallas_call(body, grid=(NPAD // BR,), in_specs=in_specs,
                          out_specs=out_specs, out_shape=outs)


def kernel(x, edge_index, W, b, gamma, beta, W_out, b_out):
    row = edge_index[0]
    col = edge_index[1]
    pr = (jnp.arange(EPAD - E, dtype=jnp.int32) * 37) % N
    pc = N + (jnp.arange(EPAD - E, dtype=jnp.int32) % (NPAD - N))
    rowp = jnp.concatenate([row, pr])
    colp = jnp.concatenate([col, pc])
    rp2 = rowp * 2
    rows2 = jnp.concatenate([rp2, rp2 + 1])
    cols2 = colp
    xp = jnp.pad(x, ((0, NPAD - N), (0, 0)))
    Wt = jnp.transpose(W, (0, 2, 1))
    scale = 1.0 / jnp.sqrt(1.0 + EPS)
    gs = gamma * scale
    b8 = jnp.broadcast_to(b[:, None, :], (L, 8, D))
    gs8 = jnp.broadcast_to(gs[:, None, :], (L, 8, D))
    beta8 = jnp.broadcast_to(beta[:, None, :], (L, 8, D))
    wof = jnp.zeros((D, 128), jnp.float32).at[:, :OUT].set(W_out.T)
    bof = jnp.broadcast_to(jnp.pad(b_out, (0, 128 - OUT))[None, :], (8, 128))

    ones128 = jnp.ones((CH, HW), jnp.float32)
    zeroshw = jnp.zeros((ZR, HW), jnp.float32)

    deg16 = _sc_degree(colp, ones128, zeroshw)

    par_spec = _full_spec((8, D))
    deg_spec = _row_spec(HW)

    def seg(z):
        return _sc_segsum(z.reshape(2 * NPAD, HW), rows2, cols2, zeroshw)

    z = _pcall(_tc_first_body,
               [_row_spec(), _full_spec((D, D)), deg_spec],
               (D,))(xp, Wt[0], deg16)
    h = xp
    for i in range(1, L):
        accs = seg(z)
        h, z = _pcall(_tc_mid_body,
                      [_accl_spec, _accr_spec, _row_spec(), _row_spec(),
                       _full_spec((D, D)), deg_spec,
                       par_spec, par_spec, par_spec], (D, D))(
            accs, accs, z, h, Wt[i], deg16,
            b8[i - 1], gs8[i - 1], beta8[i - 1])
    accs = seg(z)
    z = _pcall(_tc_last_body,
               [_accl_spec, _accr_spec, _row_spec(), _row_spec(), deg_spec,
                par_spec, par_spec, par_spec], (D,))(
        accs, accs, z, h, deg16, b8[L - 1], gs8[L - 1], beta8[L - 1])
    accs = seg(z)
    dxp = _pcall(_tc_out_body,
                 [_accl_spec, _accr_spec, _row_spec(), deg_spec,
                  _full_spec((D, 128)), _full_spec((8, 128))], (128,))(
        accs, accs, z, deg16, wof, bof)
    return dxp[:N, :OUT]


# speedup vs baseline: 12.2533x; 1.0493x over previous
import functools

import jax
import jax.numpy as jnp
from jax import lax
from jax.experimental import pallas as pl
from jax.experimental.pallas import tpu as pltpu
from jax.experimental.pallas import tpu_sc as plsc

N = 10000
NPAD = 10240
D = 256
HW = D // 2
L = 5
OUT = 3
EPS = 1e-5
BR = 1024

NC = 2
NS = 16
CH = 128
E = 160000
EPT = 10240
NCHUNK = EPT // CH
EPAD = EPT * NS
ZR = 16

_mesh = plsc.VectorSubcoreMesh(core_axis_name="c", subcore_axis_name="s",
                               num_cores=NC, num_subcores=NS)


EPW = EPAD // (NC * NS)
NCHD = EPW // CH


@functools.partial(
    pl.kernel,
    out_type=jax.ShapeDtypeStruct((2 * NPAD, HW), jnp.float32),
    mesh=_mesh,
    scratch_types=[
        pltpu.VMEM((CH,), jnp.int32),
        pltpu.VMEM((CH,), jnp.int32),
        pltpu.VMEM((CH, HW), jnp.float32),
        pltpu.VMEM((ZR, HW), jnp.float32),
        pltpu.VMEM_SHARED((NPAD, HW), jnp.float32),
        pltpu.SemaphoreType.DMA,
        pltpu.SemaphoreType.DMA,
    ],
)
def _sc_degree(colp_hbm, ones_hbm, zeros_hbm, deg_hbm,
               iA, iB, ones, zbuf, spdeg, sA, sB):
    s = lax.axis_index("c")
    t = lax.axis_index("s")
    pltpu.sync_copy(ones_hbm, ones)
    pltpu.sync_copy(zeros_hbm, zbuf)
    rpt = NPAD // NS
    for j in range(rpt // ZR):
        pltpu.sync_copy(zbuf, spdeg.at[pl.ds(t * rpt + j * ZR, ZR)])
    plsc.subcore_barrier()
    base = (s * NS + t) * EPW

    def wait_scatter(sem):
        pltpu.make_async_copy(ones_hbm, ones, sem).wait()

    pltpu.sync_copy(colp_hbm.at[pl.ds(base, CH)], iA)
    pltpu.async_copy(ones, spdeg.at[iA], sA, add=True)
    pltpu.sync_copy(colp_hbm.at[pl.ds(base + CH, CH)], iB)
    pltpu.async_copy(ones, spdeg.at[iB], sB, add=True)

    def pair(k2, _):
        a = 2 * k2
        wait_scatter(sA)
        pltpu.sync_copy(colp_hbm.at[pl.ds(base + (a + 2) * CH, CH)], iA)
        pltpu.async_copy(ones, spdeg.at[iA], sA, add=True)
        wait_scatter(sB)
        pltpu.sync_copy(colp_hbm.at[pl.ds(base + (a + 3) * CH, CH)], iB)
        pltpu.async_copy(ones, spdeg.at[iB], sB, add=True)
        return 0

    lax.fori_loop(0, NCHD // 2 - 1, pair, 0)
    wait_scatter(sA)
    wait_scatter(sB)
    plsc.subcore_barrier()
    pltpu.sync_copy(spdeg.at[pl.ds(t * rpt, rpt)],
                    deg_hbm.at[pl.ds(s * NPAD + t * rpt, rpt)])


@functools.partial(
    pl.kernel,
    out_type=jax.ShapeDtypeStruct((2 * NPAD, HW), jnp.float32),
    mesh=_mesh,
    scratch_types=[
        pltpu.VMEM((CH,), jnp.int32),
        pltpu.VMEM((CH,), jnp.int32),
        pltpu.VMEM((CH,), jnp.int32),
        pltpu.VMEM((CH,), jnp.int32),
        pltpu.VMEM((CH, HW), jnp.float32),
        pltpu.VMEM((CH, HW), jnp.float32),
        pltpu.VMEM((ZR, HW), jnp.float32),
        pltpu.VMEM_SHARED((NPAD, HW), jnp.float32),
        pltpu.SemaphoreType.DMA,
        pltpu.SemaphoreType.DMA,
        pltpu.SemaphoreType.DMA,
        pltpu.SemaphoreType.DMA,
    ],
)
def _sc_segsum(zs_hbm, rows2_hbm, cols2_hbm, zeros_hbm, accs_hbm,
               iAr, iAc, iBr, iBc, gA, gB, zbuf, spacc,
               sGA, sGB, sSA, sSB):
    s = lax.axis_index("c")
    t = lax.axis_index("s")
    pltpu.sync_copy(zeros_hbm, zbuf)
    rpt = NPAD // NS
    for j in range(rpt // ZR):
        pltpu.sync_copy(zbuf, spacc.at[pl.ds(t * rpt + j * ZR, ZR)])
    plsc.subcore_barrier()
    rbase = (s * NS + t) * EPT
    cbase = t * EPT

    def load_idx(k, ir, ic):
        pltpu.sync_copy(rows2_hbm.at[pl.ds(rbase + k * CH, CH)], ir)
        pltpu.sync_copy(cols2_hbm.at[pl.ds(cbase + k * CH, CH)], ic)

    def wait_dma(g, sem):
        pltpu.make_async_copy(zs_hbm.at[pl.ds(0, CH)], g, sem).wait()

    load_idx(0, iAr, iAc)
    pltpu.async_copy(zs_hbm.at[iAr], gA, sGA)
    load_idx(1, iBr, iBc)
    pltpu.async_copy(zs_hbm.at[iBr], gB, sGB)

    def pair(k2, _):
        a = 2 * k2
        wait_dma(gA, sGA)
        pltpu.async_copy(gA, spacc.at[iAc], sSA, add=True)
        load_idx(a + 2, iAr, iAc)
        wait_dma(gA, sSA)
        pltpu.async_copy(zs_hbm.at[iAr], gA, sGA)
        wait_dma(gB, sGB)
        pltpu.async_copy(gB, spacc.at[iBc], sSB, add=True)
        load_idx(a + 3, iBr, iBc)
        wait_dma(gB, sSB)
        pltpu.async_copy(zs_hbm.at[iBr], gB, sGB)
        return 0

    lax.fori_loop(0, NCHUNK // 2 - 1, pair, 0)
    wait_dma(gA, sGA)
    pltpu.sync_copy(gA, spacc.at[iAc], add=True)
    wait_dma(gB, sGB)
    pltpu.sync_copy(gB, spacc.at[iBc], add=True)
    plsc.subcore_barrier()
    pltpu.sync_copy(spacc.at[pl.ds(t * rpt, rpt)],
                    accs_hbm.at[pl.ds(s * NPAD + t * rpt, rpt)])


def _dis(dga_ref, dgb_ref):
    return lax.rsqrt(dga_ref[:, 0:1] + dgb_ref[:, 0:1] + 1.0)


def _tc_first_body(h_ref, w_ref, dga_ref, dgb_ref, z_ref):
    z_ref[...] = _dis(dga_ref, dgb_ref) * jnp.dot(
        h_ref[...], w_ref[...], preferred_element_type=jnp.float32)


def _epilogue(accl_ref, accr_ref, zp_ref, dga_ref, dgb_ref,
              b_ref, gs_ref, beta_ref):
    dis = _dis(dga_ref, dgb_ref)
    zp = zp_ref[...]
    conv = dis * jnp.concatenate(
        [accl_ref[...] + zp[:, 0:HW], accr_ref[...] + zp[:, HW:D]], axis=1)
    conv = conv + b_ref[0:1, :]
    a = jnp.where(conv > 0, conv, 0.2 * conv)
    return a * gs_ref[0:1, :] + beta_ref[0:1, :], dis


def _tc_mid_body(accl_ref, accr_ref, zp_ref, hp_ref, w_ref, dga_ref,
                 dgb_ref, b_ref, gs_ref, beta_ref, h_ref, z_ref):
    a, dis = _epilogue(accl_ref, accr_ref, zp_ref, dga_ref, dgb_ref,
                       b_ref, gs_ref, beta_ref)
    h = a + hp_ref[...]
    h_ref[...] = h
    z_ref[...] = dis * jnp.dot(h, w_ref[...],
                               preferred_element_type=jnp.float32)


def _tc_last_body(accl_ref, accr_ref, zp_ref, hp_ref, dga_ref, dgb_ref,
                  b_ref, gs_ref, beta_ref, z_ref):
    a, dis = _epilogue(accl_ref, accr_ref, zp_ref, dga_ref, dgb_ref,
                       b_ref, gs_ref, beta_ref)
    z_ref[...] = dis * (a + hp_ref[...])


def _tc_out_body(accl_ref, accr_ref, zp_ref, dga_ref, dgb_ref,
                 w_ref, b_ref, dx_ref):
    dis = _dis(dga_ref, dgb_ref)
    zp = zp_ref[...]
    agg = dis * jnp.concatenate(
        [accl_ref[...] + zp[:, 0:HW], accr_ref[...] + zp[:, HW:D]], axis=1)
    dx_ref[...] = jnp.dot(agg, w_ref[...],
                          preferred_element_type=jnp.float32) + b_ref[0:1, :]


def _row_spec(width=D):
    return pl.BlockSpec((BR, width), lambda i: (i, 0))


_accl_spec = pl.BlockSpec((BR, HW), lambda i: (i, 0))
_accr_spec = pl.BlockSpec((BR, HW), lambda i: (i + NPAD // BR, 0))


def _full_spec(shape):
    return pl.BlockSpec(shape, lambda i: (0, 0))


def _pcall(body, in_specs, out_widths):
    outs = tuple(jax.ShapeDtypeStruct((NPAD, w), jnp.float32)
                 for w in out_widths)
    out_specs = tuple(_row_spec(w) for w in out_widths)
    if len(out_widths) == 1:
        outs, out_specs = outs[0], out_specs[0]
    return pl.pallas_call(body, grid=(NPAD // BR,), in_specs=in_specs,
                          out_specs=out_specs, out_shape=outs)


def kernel(x, edge_index, W, b, gamma, beta, W_out, b_out):
    row = edge_index[0]
    col = edge_index[1]
    pr = (jnp.arange(EPAD - E, dtype=jnp.int32) * 37) % N
    pc = N + (jnp.arange(EPAD - E, dtype=jnp.int32) % (NPAD - N))
    rowp = jnp.concatenate([row, pr])
    colp = jnp.concatenate([col, pc])
    rp2 = rowp * 2
    rows2 = jnp.concatenate([rp2, rp2 + 1])
    cols2 = colp
    xp = jnp.pad(x, ((0, NPAD - N), (0, 0)))
    Wt = jnp.transpose(W, (0, 2, 1))
    scale = 1.0 / jnp.sqrt(1.0 + EPS)
    gs = gamma * scale
    b8 = jnp.broadcast_to(b[:, None, :], (L, 8, D))
    gs8 = jnp.broadcast_to(gs[:, None, :], (L, 8, D))
    beta8 = jnp.broadcast_to(beta[:, None, :], (L, 8, D))
    wof = jnp.zeros((D, 128), jnp.float32).at[:, :OUT].set(W_out.T)
    bof = jnp.broadcast_to(jnp.pad(b_out, (0, 128 - OUT))[None, :], (8, 128))

    ones128 = jnp.ones((CH, HW), jnp.float32)
    zeroshw = jnp.zeros((ZR, HW), jnp.float32)

    deg2 = _sc_degree(colp, ones128, zeroshw)

    par_spec = _full_spec((8, D))

    def seg(z):
        return _sc_segsum(z.reshape(2 * NPAD, HW), rows2, cols2, zeroshw)

    z = _pcall(_tc_first_body,
               [_row_spec(), _full_spec((D, D)), _accl_spec, _accr_spec],
               (D,))(xp, Wt[0], deg2, deg2)
    h = xp
    for i in range(1, L):
        accs = seg(z)
        h, z = _pcall(_tc_mid_body,
                      [_accl_spec, _accr_spec, _row_spec(), _row_spec(),
                       _full_spec((D, D)), _accl_spec, _accr_spec,
                       par_spec, par_spec, par_spec], (D, D))(
            accs, accs, z, h, Wt[i], deg2, deg2,
            b8[i - 1], gs8[i - 1], beta8[i - 1])
    accs = seg(z)
    z = _pcall(_tc_last_body,
               [_accl_spec, _accr_spec, _row_spec(), _row_spec(),
                _accl_spec, _accr_spec,
                par_spec, par_spec, par_spec], (D,))(
        accs, accs, z, h, deg2, deg2, b8[L - 1], gs8[L - 1], beta8[L - 1])
    accs = seg(z)
    dxp = _pcall(_tc_out_body,
                 [_accl_spec, _accr_spec, _row_spec(), _accl_spec, _accr_spec,
                  _full_spec((D, 128)), _full_spec((8, 128))], (128,))(
        accs, accs, z, deg2, deg2, wof, bof)
    return dxp[:N, :OUT]
